# baseline (speedup 1.0000x reference)
import functools

import jax
import jax.numpy as jnp
from jax import lax
from jax.experimental import pallas as pl
from jax.experimental.pallas import tpu as pltpu
from jax.experimental.pallas import tpu_sc as plsc

_T, _B, _D, _L = 20, 32, 128, 3
_TB = _T * _B
_G4 = 4 * _D
_VOCAB = 100000
_VT = 4096
_NT = (_VOCAB + _VT - 1) // _VT
_KRES = 15
_RROWS = _KRES * _VT
_NW = 6
_WROWS = _RROWS // _NW


@functools.lru_cache(maxsize=None)
def _make_sc_gather(vocab, d, n_idx):
    info = plsc.get_sparse_core_info()
    nc, ns = info.num_cores, info.num_subcores
    nw = nc * ns
    b_per_w = 32
    n_active = n_idx // b_per_w
    assert n_idx % b_per_w == 0 and n_active <= nw
    mesh = plsc.VectorSubcoreMesh(core_axis_name="c", subcore_axis_name="s")

    @functools.partial(
        pl.kernel,
        mesh=mesh,
        out_type=jax.ShapeDtypeStruct((n_idx, d), jnp.float32),
        scratch_types=[
            pltpu.VMEM((b_per_w,), jnp.int32),
            pltpu.VMEM((b_per_w, d), jnp.float32),
            pltpu.SemaphoreType.DMA,
        ],
    )
    def gather_k(table_hbm, idx_hbm, out_hbm, idx_v, rows_v, sem):
        wid = lax.axis_index("s") * nc + lax.axis_index("c")

        @pl.when(wid < n_active)
        def _():
            base = wid * b_per_w
            pltpu.sync_copy(idx_hbm.at[pl.ds(base, b_per_w)], idx_v)
            pltpu.async_copy(table_hbm.at[idx_v], rows_v, sem).wait()
            pltpu.sync_copy(rows_v, out_hbm.at[pl.ds(base, b_per_w)])

    return gather_k


def _w_copy(i, fcw_hbm, wf_ref, semw):
    return pltpu.make_async_copy(
        fcw_hbm.at[pl.ds(i * _WROWS, _WROWS)],
        wf_ref.at[pl.ds(i * _WROWS, _WROWS)],
        semw.at[i])


def _fused_body(x_ref, h0_ref, c0_ref,
                wih0, whh0, bih0, bhh0,
                wih1, whh1, bih1, bhh1,
                wih2, whh2, bih2, bhh2,
                fcw_hbm, wstream_ref, fcb_ref,
                logits_ref, ht_ref, ct_ref,
                ys_ref, gx_ref, wf_ref, semw):
    step = pl.program_id(0)

    @pl.when(step == 0)
    def _prologue():
        for i in range(_NW):
            _w_copy(i, fcw_hbm, wf_ref, semw).start()

        params = ((wih0, whh0, bih0, bhh0),
                  (wih1, whh1, bih1, bhh1),
                  (wih2, whh2, bih2, bhh2))
        dn = (((1,), (1,)), ((), ()))
        gx_ref[:] = (
            lax.dot_general(x_ref[:], wih0[:], dn,
                            preferred_element_type=jnp.float32)
            + bih0[:] + bhh0[:]
        )
        hs = [h0_ref[l] for l in range(_L)]
        cs = [c0_ref[l] for l in range(_L)]
        for s in range(_T + _L - 1):
            nxt = {}
            for l in range(_L):
                t = s - l
                if not 0 <= t < _T:
                    continue
                wih, whh, bih, bhh = params[l]
                rec = lax.dot_general(hs[l], whh[:], dn,
                                      preferred_element_type=jnp.float32)
                if l == 0:
                    gates = gx_ref[t * _B:(t + 1) * _B, :] + rec
                else:
                    gates = (lax.dot_general(hs[l - 1], wih[:], dn,
                                             preferred_element_type=jnp.float32)
                             + bih[:] + bhh[:] + rec)
                i_g = jax.nn.sigmoid(gates[:, :_D])
                f_g = jax.nn.sigmoid(gates[:, _D:2 * _D])
                g_g = jnp.tanh(gates[:, 2 * _D:3 * _D])
                o_g = jax.nn.sigmoid(gates[:, 3 * _D:])
                c_n = f_g * cs[l] + i_g * g_g
                h_n = o_g * jnp.tanh(c_n)
                nxt[l] = (h_n, c_n)
                if l == _L - 1:
                    ys_ref[t * _B:(t + 1) * _B, :] = h_n
            for l, (h_n, c_n) in nxt.items():
                hs[l], cs[l] = h_n, c_n
        for l in range(_L):
            ht_ref[l] = hs[l]
            ct_ref[l] = cs[l]

        for i in range(_NW):
            _w_copy(i, fcw_hbm, wf_ref, semw).wait()

    dnp = (((1,), (1,)), ((), ()))

    @pl.when(step < _KRES)
    def _resident_tile():
        base = pl.multiple_of(step * _VT, _VT)
        wv = wf_ref[pl.ds(base, _VT), :]
        logits_ref[:] = lax.dot_general(
            ys_ref[:], wv, dnp,
            preferred_element_type=jnp.float32) + fcb_ref[:]

    @pl.when(step >= _KRES)
    def _streamed_tile():
        logits_ref[:] = lax.dot_general(
            ys_ref[:], wstream_ref[:], dnp,
            preferred_element_type=jnp.float32) + fcb_ref[:]


def _run_fused(xs, h0, c0, ws, fc_w, fcb_pad):
    vfull = pl.BlockSpec(memory_space=pltpu.MemorySpace.VMEM)
    hspec = pl.BlockSpec(memory_space=pltpu.MemorySpace.HBM)
    out_shapes = (
        jax.ShapeDtypeStruct((_TB, _VOCAB), jnp.float32),
        jax.ShapeDtypeStruct((_L, _B, _D), jnp.float32),
        jax.ShapeDtypeStruct((_L, _B, _D), jnp.float32),
    )
    return pl.pallas_call(
        _fused_body,
        grid=(_NT,),
        in_specs=[vfull] * 15 + [
            hspec,
            pl.BlockSpec((_VT, _D),
                         lambda i: (jnp.maximum(i, _KRES), 0)),
            pl.BlockSpec((1, _VT), lambda i: (0, i)),
        ],
        out_specs=(
            pl.BlockSpec((_TB, _VT), lambda i: (0, i)),
            pl.BlockSpec((_L, _B, _D), lambda i: (0, 0, 0)),
            pl.BlockSpec((_L, _B, _D), lambda i: (0, 0, 0)),
        ),
        out_shape=out_shapes,
        scratch_shapes=[
            pltpu.VMEM((_TB, _D), jnp.float32),
            pltpu.VMEM((_TB, _G4), jnp.float32),
            pltpu.VMEM((_RROWS, _D), jnp.float32),
            pltpu.SemaphoreType.DMA((_NW,)),
        ],
        compiler_params=pltpu.CompilerParams(
            dimension_semantics=("arbitrary",)),
    )(xs, h0, c0, *ws, fc_w, fc_w, fcb_pad)


def kernel(x, h0, c0, emb,
           W_ih0, W_hh0, b_ih0, b_hh0,
           W_ih1, W_hh1, b_ih1, b_hh1,
           W_ih2, W_hh2, b_ih2, b_hh2,
           fc_W, fc_b):
    vocab = emb.shape[0]
    idx = x.reshape(_TB)
    gathered = _make_sc_gather(vocab, _D, _TB)(emb, idx)

    ws = (W_ih0, W_hh0, b_ih0.reshape(1, _G4), b_hh0.reshape(1, _G4),
          W_ih1, W_hh1, b_ih1.reshape(1, _G4), b_hh1.reshape(1, _G4),
          W_ih2, W_hh2, b_ih2.reshape(1, _G4), b_hh2.reshape(1, _G4))
    logits, ht, ct = _run_fused(gathered, h0, c0, ws, fc_W,
                                fc_b.reshape(1, vocab))
    return logits.reshape(_T, _B, vocab), (ht, ct)


# speedup vs baseline: 1.0014x; 1.0014x over previous
import functools

import jax
import jax.numpy as jnp
from jax import lax
from jax.experimental import pallas as pl
from jax.experimental.pallas import tpu as pltpu
from jax.experimental.pallas import tpu_sc as plsc

_T, _B, _D, _L = 20, 32, 128, 3
_TB = _T * _B
_G4 = 4 * _D
_VOCAB = 100000
_VT = 4096
_NT = (_VOCAB + _VT - 1) // _VT
_KRES = 15
_RROWS = _KRES * _VT
_NW = 2
_WROWS = _RROWS // _NW


@functools.lru_cache(maxsize=None)
def _make_sc_gather(vocab, d, n_idx):
    info = plsc.get_sparse_core_info()
    nc, ns = info.num_cores, info.num_subcores
    nw = nc * ns
    b_per_w = 32
    n_active = n_idx // b_per_w
    assert n_idx % b_per_w == 0 and n_active <= nw
    mesh = plsc.VectorSubcoreMesh(core_axis_name="c", subcore_axis_name="s")

    @functools.partial(
        pl.kernel,
        mesh=mesh,
        out_type=jax.ShapeDtypeStruct((n_idx, d), jnp.float32),
        scratch_types=[
            pltpu.VMEM((b_per_w,), jnp.int32),
            pltpu.VMEM((b_per_w, d), jnp.float32),
            pltpu.SemaphoreType.DMA,
        ],
    )
    def gather_k(table_hbm, idx_hbm, out_hbm, idx_v, rows_v, sem):
        wid = lax.axis_index("s") * nc + lax.axis_index("c")

        @pl.when(wid < n_active)
        def _():
            base = wid * b_per_w
            pltpu.sync_copy(idx_hbm.at[pl.ds(base, b_per_w)], idx_v)
            pltpu.async_copy(table_hbm.at[idx_v], rows_v, sem).wait()
            pltpu.sync_copy(rows_v, out_hbm.at[pl.ds(base, b_per_w)])

    return gather_k


def _w_copy(i, fcw_hbm, wf_ref, semw):
    return pltpu.make_async_copy(
        fcw_hbm.at[pl.ds(i * _WROWS, _WROWS)],
        wf_ref.at[pl.ds(i * _WROWS, _WROWS)],
        semw.at[i])


def _fused_body(x_ref, h0_ref, c0_ref,
                wih0, whh0, bih0, bhh0,
                wih1, whh1, bih1, bhh1,
                wih2, whh2, bih2, bhh2,
                fcw_hbm, wstream_ref, fcb_ref,
                logits_ref, ht_ref, ct_ref,
                ys_ref, gx_ref, wf_ref, semw):
    step = pl.program_id(0)

    @pl.when(step == 0)
    def _prologue():
        for i in range(_NW):
            _w_copy(i, fcw_hbm, wf_ref, semw).start()

        params = ((wih0, whh0, bih0, bhh0),
                  (wih1, whh1, bih1, bhh1),
                  (wih2, whh2, bih2, bhh2))
        dn = (((1,), (1,)), ((), ()))
        gx_ref[:] = (
            lax.dot_general(x_ref[:], wih0[:], dn,
                            preferred_element_type=jnp.float32)
            + bih0[:] + bhh0[:]
        )
        hs = [h0_ref[l] for l in range(_L)]
        cs = [c0_ref[l] for l in range(_L)]
        for s in range(_T + _L - 1):
            nxt = {}
            for l in range(_L):
                t = s - l
                if not 0 <= t < _T:
                    continue
                wih, whh, bih, bhh = params[l]
                rec = lax.dot_general(hs[l], whh[:], dn,
                                      preferred_element_type=jnp.float32)
                if l == 0:
                    gates = gx_ref[t * _B:(t + 1) * _B, :] + rec
                else:
                    gates = (lax.dot_general(hs[l - 1], wih[:], dn,
                                             preferred_element_type=jnp.float32)
                             + bih[:] + bhh[:] + rec)
                i_g = jax.nn.sigmoid(gates[:, :_D])
                f_g = jax.nn.sigmoid(gates[:, _D:2 * _D])
                g_g = jnp.tanh(gates[:, 2 * _D:3 * _D])
                o_g = jax.nn.sigmoid(gates[:, 3 * _D:])
                c_n = f_g * cs[l] + i_g * g_g
                h_n = o_g * jnp.tanh(c_n)
                nxt[l] = (h_n, c_n)
                if l == _L - 1:
                    ys_ref[t * _B:(t + 1) * _B, :] = h_n
            for l, (h_n, c_n) in nxt.items():
                hs[l], cs[l] = h_n, c_n
        for l in range(_L):
            ht_ref[l] = hs[l]
            ct_ref[l] = cs[l]

        for i in range(_NW):
            _w_copy(i, fcw_hbm, wf_ref, semw).wait()

    dnp = (((1,), (1,)), ((), ()))

    @pl.when(step < _KRES)
    def _resident_tile():
        base = pl.multiple_of(step * _VT, _VT)
        wv = wf_ref[pl.ds(base, _VT), :]
        logits_ref[:] = lax.dot_general(
            ys_ref[:], wv, dnp,
            preferred_element_type=jnp.float32) + fcb_ref[:]

    @pl.when(step >= _KRES)
    def _streamed_tile():
        logits_ref[:] = lax.dot_general(
            ys_ref[:], wstream_ref[:], dnp,
            preferred_element_type=jnp.float32) + fcb_ref[:]


def _run_fused(xs, h0, c0, ws, fc_w, fcb_pad):
    vfull = pl.BlockSpec(memory_space=pltpu.MemorySpace.VMEM)
    hspec = pl.BlockSpec(memory_space=pltpu.MemorySpace.HBM)
    out_shapes = (
        jax.ShapeDtypeStruct((_TB, _VOCAB), jnp.float32),
        jax.ShapeDtypeStruct((_L, _B, _D), jnp.float32),
        jax.ShapeDtypeStruct((_L, _B, _D), jnp.float32),
    )
    return pl.pallas_call(
        _fused_body,
        grid=(_NT,),
        in_specs=[vfull] * 15 + [
            hspec,
            pl.BlockSpec((_VT, _D),
                         lambda i: (jnp.maximum(i, _KRES), 0)),
            pl.BlockSpec((1, _VT), lambda i: (0, i)),
        ],
        out_specs=(
            pl.BlockSpec((_TB, _VT), lambda i: (0, i)),
            pl.BlockSpec((_L, _B, _D), lambda i: (0, 0, 0)),
            pl.BlockSpec((_L, _B, _D), lambda i: (0, 0, 0)),
        ),
        out_shape=out_shapes,
        scratch_shapes=[
            pltpu.VMEM((_TB, _D), jnp.float32),
            pltpu.VMEM((_TB, _G4), jnp.float32),
            pltpu.VMEM((_RROWS, _D), jnp.float32),
            pltpu.SemaphoreType.DMA((_NW,)),
        ],
        compiler_params=pltpu.CompilerParams(
            dimension_semantics=("arbitrary",)),
    )(xs, h0, c0, *ws, fc_w, fc_w, fcb_pad)


def kernel(x, h0, c0, emb,
           W_ih0, W_hh0, b_ih0, b_hh0,
           W_ih1, W_hh1, b_ih1, b_hh1,
           W_ih2, W_hh2, b_ih2, b_hh2,
           fc_W, fc_b):
    vocab = emb.shape[0]
    idx = x.reshape(_TB)
    gathered = _make_sc_gather(vocab, _D, _TB)(emb, idx)

    ws = (W_ih0, W_hh0, b_ih0.reshape(1, _G4), b_hh0.reshape(1, _G4),
          W_ih1, W_hh1, b_ih1.reshape(1, _G4), b_hh1.reshape(1, _G4),
          W_ih2, W_hh2, b_ih2.reshape(1, _G4), b_hh2.reshape(1, _G4))
    logits, ht, ct = _run_fused(gathered, h0, c0, ws, fc_W,
                                fc_b.reshape(1, vocab))
    return logits.reshape(_T, _B, vocab), (ht, ct)


# speedup vs baseline: 1.0305x; 1.0290x over previous
import functools

import jax
import jax.numpy as jnp
from jax import lax
from jax.experimental import pallas as pl
from jax.experimental.pallas import tpu as pltpu
from jax.experimental.pallas import tpu_sc as plsc

_T, _B, _D, _L = 20, 32, 128, 3
_TB = _T * _B
_G4 = 4 * _D
_VOCAB = 100000
_VT = 4096
_NT = (_VOCAB + _VT - 1) // _VT
_KRES = 15
_RROWS = _KRES * _VT
_NW = _KRES
_WROWS = _VT


@functools.lru_cache(maxsize=None)
def _make_sc_gather(vocab, d, n_idx):
    info = plsc.get_sparse_core_info()
    nc, ns = info.num_cores, info.num_subcores
    nw = nc * ns
    b_per_w = 32
    n_active = n_idx // b_per_w
    assert n_idx % b_per_w == 0 and n_active <= nw
    mesh = plsc.VectorSubcoreMesh(core_axis_name="c", subcore_axis_name="s")

    @functools.partial(
        pl.kernel,
        mesh=mesh,
        out_type=jax.ShapeDtypeStruct((n_idx, d), jnp.float32),
        scratch_types=[
            pltpu.VMEM((b_per_w,), jnp.int32),
            pltpu.VMEM((b_per_w, d), jnp.float32),
            pltpu.SemaphoreType.DMA,
        ],
    )
    def gather_k(table_hbm, idx_hbm, out_hbm, idx_v, rows_v, sem):
        wid = lax.axis_index("s") * nc + lax.axis_index("c")

        @pl.when(wid < n_active)
        def _():
            base = wid * b_per_w
            pltpu.sync_copy(idx_hbm.at[pl.ds(base, b_per_w)], idx_v)
            pltpu.async_copy(table_hbm.at[idx_v], rows_v, sem).wait()
            pltpu.sync_copy(rows_v, out_hbm.at[pl.ds(base, b_per_w)])

    return gather_k


def _w_copy(i, fcw_hbm, wf_ref, semw):
    return pltpu.make_async_copy(
        fcw_hbm.at[pl.ds(i * _WROWS, _WROWS)],
        wf_ref.at[pl.ds(i * _WROWS, _WROWS)],
        semw.at[i])


def _fused_body(x_ref, h0_ref, c0_ref,
                wih0, whh0, bih0, bhh0,
                wih1, whh1, bih1, bhh1,
                wih2, whh2, bih2, bhh2,
                fcw_hbm, wstream_ref, fcb_ref,
                logits_ref, ht_ref, ct_ref,
                ys_ref, gx_ref, wf_ref, semw):
    step = pl.program_id(0)

    @pl.when(step == 0)
    def _prologue():
        for i in range(_NW):
            _w_copy(i, fcw_hbm, wf_ref, semw).start()

        params = ((wih0, whh0, bih0, bhh0),
                  (wih1, whh1, bih1, bhh1),
                  (wih2, whh2, bih2, bhh2))
        dn = (((1,), (1,)), ((), ()))
        gx_ref[:] = (
            lax.dot_general(x_ref[:], wih0[:], dn,
                            preferred_element_type=jnp.float32)
            + bih0[:] + bhh0[:]
        )
        hs = [h0_ref[l] for l in range(_L)]
        cs = [c0_ref[l] for l in range(_L)]
        for s in range(_T + _L - 1):
            nxt = {}
            for l in range(_L):
                t = s - l
                if not 0 <= t < _T:
                    continue
                wih, whh, bih, bhh = params[l]
                rec = lax.dot_general(hs[l], whh[:], dn,
                                      preferred_element_type=jnp.float32)
                if l == 0:
                    gates = gx_ref[t * _B:(t + 1) * _B, :] + rec
                else:
                    gates = (lax.dot_general(hs[l - 1], wih[:], dn,
                                             preferred_element_type=jnp.float32)
                             + bih[:] + bhh[:] + rec)
                i_g = jax.nn.sigmoid(gates[:, :_D])
                f_g = jax.nn.sigmoid(gates[:, _D:2 * _D])
                g_g = jnp.tanh(gates[:, 2 * _D:3 * _D])
                o_g = jax.nn.sigmoid(gates[:, 3 * _D:])
                c_n = f_g * cs[l] + i_g * g_g
                h_n = o_g * jnp.tanh(c_n)
                nxt[l] = (h_n, c_n)
                if l == _L - 1:
                    ys_ref[t * _B:(t + 1) * _B, :] = h_n
            for l, (h_n, c_n) in nxt.items():
                hs[l], cs[l] = h_n, c_n
        for l in range(_L):
            ht_ref[l] = hs[l]
            ct_ref[l] = cs[l]

    dnp = (((1,), (1,)), ((), ()))

    @pl.when(step < _KRES)
    def _resident_tile():
        for j in range(_KRES):
            @pl.when(step == j)
            def _(j=j):
                _w_copy(j, fcw_hbm, wf_ref, semw).wait()
        base = pl.multiple_of(step * _VT, _VT)
        wv = wf_ref[pl.ds(base, _VT), :]
        logits_ref[:] = lax.dot_general(
            ys_ref[:], wv, dnp,
            preferred_element_type=jnp.float32) + fcb_ref[:]

    @pl.when(step >= _KRES)
    def _streamed_tile():
        logits_ref[:] = lax.dot_general(
            ys_ref[:], wstream_ref[:], dnp,
            preferred_element_type=jnp.float32) + fcb_ref[:]


def _run_fused(xs, h0, c0, ws, fc_w, fcb_pad):
    vfull = pl.BlockSpec(memory_space=pltpu.MemorySpace.VMEM)
    hspec = pl.BlockSpec(memory_space=pltpu.MemorySpace.HBM)
    out_shapes = (
        jax.ShapeDtypeStruct((_TB, _VOCAB), jnp.float32),
        jax.ShapeDtypeStruct((_L, _B, _D), jnp.float32),
        jax.ShapeDtypeStruct((_L, _B, _D), jnp.float32),
    )
    return pl.pallas_call(
        _fused_body,
        grid=(_NT,),
        in_specs=[vfull] * 15 + [
            hspec,
            pl.BlockSpec((_VT, _D),
                         lambda i: (jnp.maximum(i, _KRES), 0)),
            pl.BlockSpec((1, _VT), lambda i: (0, i)),
        ],
        out_specs=(
            pl.BlockSpec((_TB, _VT), lambda i: (0, i)),
            pl.BlockSpec((_L, _B, _D), lambda i: (0, 0, 0)),
            pl.BlockSpec((_L, _B, _D), lambda i: (0, 0, 0)),
        ),
        out_shape=out_shapes,
        scratch_shapes=[
            pltpu.VMEM((_TB, _D), jnp.float32),
            pltpu.VMEM((_TB, _G4), jnp.float32),
            pltpu.VMEM((_RROWS, _D), jnp.float32),
            pltpu.SemaphoreType.DMA((_NW,)),
        ],
        compiler_params=pltpu.CompilerParams(
            dimension_semantics=("arbitrary",)),
    )(xs, h0, c0, *ws, fc_w, fc_w, fcb_pad)


def kernel(x, h0, c0, emb,
           W_ih0, W_hh0, b_ih0, b_hh0,
           W_ih1, W_hh1, b_ih1, b_hh1,
           W_ih2, W_hh2, b_ih2, b_hh2,
           fc_W, fc_b):
    vocab = emb.shape[0]
    idx = x.reshape(_TB)
    gathered = _make_sc_gather(vocab, _D, _TB)(emb, idx)

    ws = (W_ih0, W_hh0, b_ih0.reshape(1, _G4), b_hh0.reshape(1, _G4),
          W_ih1, W_hh1, b_ih1.reshape(1, _G4), b_hh1.reshape(1, _G4),
          W_ih2, W_hh2, b_ih2.reshape(1, _G4), b_hh2.reshape(1, _G4))
    logits, ht, ct = _run_fused(gathered, h0, c0, ws, fc_W,
                                fc_b.reshape(1, vocab))
    return logits.reshape(_T, _B, vocab), (ht, ct)


# speedup vs baseline: 1.0339x; 1.0033x over previous
import functools

import jax
import jax.numpy as jnp
from jax import lax
from jax.experimental import pallas as pl
from jax.experimental.pallas import tpu as pltpu
from jax.experimental.pallas import tpu_sc as plsc

_T, _B, _D, _L = 20, 32, 128, 3
_TB = _T * _B
_G4 = 4 * _D
_VOCAB = 100000
_VT = 4096
_NT = (_VOCAB + _VT - 1) // _VT
_RING = 15
_RROWS = _RING * _VT
_REM = _VOCAB - (_NT - 1) * _VT


@functools.lru_cache(maxsize=None)
def _make_sc_gather(vocab, d, n_idx):
    info = plsc.get_sparse_core_info()
    nc, ns = info.num_cores, info.num_subcores
    nw = nc * ns
    b_per_w = 32
    n_active = n_idx // b_per_w
    assert n_idx % b_per_w == 0 and n_active <= nw
    mesh = plsc.VectorSubcoreMesh(core_axis_name="c", subcore_axis_name="s")

    @functools.partial(
        pl.kernel,
        mesh=mesh,
        out_type=jax.ShapeDtypeStruct((n_idx, d), jnp.float32),
        scratch_types=[
            pltpu.VMEM((b_per_w,), jnp.int32),
            pltpu.VMEM((b_per_w, d), jnp.float32),
            pltpu.SemaphoreType.DMA,
        ],
    )
    def gather_k(table_hbm, idx_hbm, out_hbm, idx_v, rows_v, sem):
        wid = lax.axis_index("s") * nc + lax.axis_index("c")

        @pl.when(wid < n_active)
        def _():
            base = wid * b_per_w
            pltpu.sync_copy(idx_hbm.at[pl.ds(base, b_per_w)], idx_v)
            pltpu.async_copy(table_hbm.at[idx_v], rows_v, sem).wait()
            pltpu.sync_copy(rows_v, out_hbm.at[pl.ds(base, b_per_w)])

    return gather_k


def _w_copy(i, fcw_hbm, wf_ref, semw):
    rows = _VT if i < _NT - 1 else _REM
    return pltpu.make_async_copy(
        fcw_hbm.at[pl.ds(i * _VT, rows)],
        wf_ref.at[pl.ds((i % _RING) * _VT, rows)],
        semw.at[i])


def _fused_body(x_ref, h0_ref, c0_ref,
                wih0, whh0, bih0, bhh0,
                wih1, whh1, bih1, bhh1,
                wih2, whh2, bih2, bhh2,
                fcw_hbm, fcb_ref,
                logits_ref, ht_ref, ct_ref,
                ys_ref, gx_ref, wf_ref, semw):
    step = pl.program_id(0)

    @pl.when(step == 0)
    def _prologue():
        for i in range(_RING):
            _w_copy(i, fcw_hbm, wf_ref, semw).start()

        params = ((wih0, whh0, bih0, bhh0),
                  (wih1, whh1, bih1, bhh1),
                  (wih2, whh2, bih2, bhh2))
        dn = (((1,), (1,)), ((), ()))
        gx_ref[:] = (
            lax.dot_general(x_ref[:], wih0[:], dn,
                            preferred_element_type=jnp.float32)
            + bih0[:] + bhh0[:]
        )
        hs = [h0_ref[l] for l in range(_L)]
        cs = [c0_ref[l] for l in range(_L)]
        for s in range(_T + _L - 1):
            nxt = {}
            for l in range(_L):
                t = s - l
                if not 0 <= t < _T:
                    continue
                wih, whh, bih, bhh = params[l]
                rec = lax.dot_general(hs[l], whh[:], dn,
                                      preferred_element_type=jnp.float32)
                if l == 0:
                    gates = gx_ref[t * _B:(t + 1) * _B, :] + rec
                else:
                    gates = (lax.dot_general(hs[l - 1], wih[:], dn,
                                             preferred_element_type=jnp.float32)
                             + bih[:] + bhh[:] + rec)
                i_g = jax.nn.sigmoid(gates[:, :_D])
                f_g = jax.nn.sigmoid(gates[:, _D:2 * _D])
                g_g = jnp.tanh(gates[:, 2 * _D:3 * _D])
                o_g = jax.nn.sigmoid(gates[:, 3 * _D:])
                c_n = f_g * cs[l] + i_g * g_g
                h_n = o_g * jnp.tanh(c_n)
                nxt[l] = (h_n, c_n)
                if l == _L - 1:
                    ys_ref[t * _B:(t + 1) * _B, :] = h_n
            for l, (h_n, c_n) in nxt.items():
                hs[l], cs[l] = h_n, c_n
        for l in range(_L):
            ht_ref[l] = hs[l]
            ct_ref[l] = cs[l]

    for j in range(_NT):
        @pl.when(step == j)
        def _(j=j):
            _w_copy(j, fcw_hbm, wf_ref, semw).wait()
    for j in range(1, _NT - _RING + 1):
        @pl.when(step == j)
        def _(j=j):
            _w_copy(j + _RING - 1, fcw_hbm, wf_ref, semw).start()

    dnp = (((1,), (1,)), ((), ()))
    slot = lax.rem(step, _RING)
    base = pl.multiple_of(slot * _VT, _VT)
    wv = wf_ref[pl.ds(base, _VT), :]
    logits_ref[:] = lax.dot_general(
        ys_ref[:], wv, dnp,
        preferred_element_type=jnp.float32) + fcb_ref[:]


def _run_fused(xs, h0, c0, ws, fc_w, fcb_pad):
    vfull = pl.BlockSpec(memory_space=pltpu.MemorySpace.VMEM)
    hspec = pl.BlockSpec(memory_space=pltpu.MemorySpace.HBM)
    out_shapes = (
        jax.ShapeDtypeStruct((_TB, _VOCAB), jnp.float32),
        jax.ShapeDtypeStruct((_L, _B, _D), jnp.float32),
        jax.ShapeDtypeStruct((_L, _B, _D), jnp.float32),
    )
    return pl.pallas_call(
        _fused_body,
        grid=(_NT,),
        in_specs=[vfull] * 15 + [
            hspec,
            pl.BlockSpec((1, _VT), lambda i: (0, i)),
        ],
        out_specs=(
            pl.BlockSpec((_TB, _VT), lambda i: (0, i)),
            pl.BlockSpec((_L, _B, _D), lambda i: (0, 0, 0)),
            pl.BlockSpec((_L, _B, _D), lambda i: (0, 0, 0)),
        ),
        out_shape=out_shapes,
        scratch_shapes=[
            pltpu.VMEM((_TB, _D), jnp.float32),
            pltpu.VMEM((_TB, _G4), jnp.float32),
            pltpu.VMEM((_RROWS, _D), jnp.float32),
            pltpu.SemaphoreType.DMA((_NT,)),
        ],
        compiler_params=pltpu.CompilerParams(
            dimension_semantics=("arbitrary",)),
    )(xs, h0, c0, *ws, fc_w, fcb_pad)


def kernel(x, h0, c0, emb,
           W_ih0, W_hh0, b_ih0, b_hh0,
           W_ih1, W_hh1, b_ih1, b_hh1,
           W_ih2, W_hh2, b_ih2, b_hh2,
           fc_W, fc_b):
    vocab = emb.shape[0]
    idx = x.reshape(_TB)
    gathered = _make_sc_gather(vocab, _D, _TB)(emb, idx)

    ws = (W_ih0, W_hh0, b_ih0.reshape(1, _G4), b_hh0.reshape(1, _G4),
          W_ih1, W_hh1, b_ih1.reshape(1, _G4), b_hh1.reshape(1, _G4),
          W_ih2, W_hh2, b_ih2.reshape(1, _G4), b_hh2.reshape(1, _G4))
    logits, ht, ct = _run_fused(gathered, h0, c0, ws, fc_W,
                                fc_b.reshape(1, vocab))
    return logits.reshape(_T, _B, vocab), (ht, ct)
